# baseline (speedup 1.0000x reference)
import functools

import jax
import jax.numpy as jnp
from jax import lax
from jax.experimental import pallas as pl
from jax.experimental.pallas import tpu as pltpu

KB = 3
_NEG = -jnp.inf
_IMAX = 2**31 - 1


def _ctx_body(enc_ref, out_ref, acc_s, *, nc, inv):
    j = pl.program_id(1)

    @pl.when(j == 0)
    def _():
        acc_s[...] = jnp.zeros_like(acc_s)

    acc_s[...] += jnp.sum(enc_ref[...], axis=1)

    @pl.when(j == nc - 1)
    def _():
        out_ref[...] = acc_s[...] * inv


def _pooled_ctx(enc):
    b, seq, d = enc.shape
    bb, ch = 8, 256
    nc = seq // ch
    return pl.pallas_call(
        functools.partial(_ctx_body, nc=nc, inv=1.0 / seq),
        grid=(b // bb, nc),
        in_specs=[pl.BlockSpec((bb, ch, d), lambda i, j: (i, j, 0))],
        out_specs=pl.BlockSpec((bb, d), lambda i, j: (i, 0)),
        out_shape=jax.ShapeDtypeStruct((b, d), jnp.float32),
        scratch_shapes=[pltpu.VMEM((bb, d), jnp.float32)],
    )(enc)


def _step_body(emb_ref, ctx_ref, wh_ref, wout_ref,
               tv_ref, ti_ref, lse_ref,
               h_s, m_s, s_s, tv_s, ti_s, *, nt, vt, vocab):
    i = pl.program_id(0)

    @pl.when(i == 0)
    def _():
        x = emb_ref[...] + ctx_ref[...]
        h_s[...] = jnp.tanh(jnp.dot(x, wh_ref[...],
                                    preferred_element_type=jnp.float32))
        m_s[...] = jnp.full_like(m_s, _NEG)
        s_s[...] = jnp.zeros_like(s_s)
        tv_s[...] = jnp.full_like(tv_s, _NEG)
        ti_s[...] = jnp.zeros_like(ti_s)

    logits = jnp.dot(h_s[...], wout_ref[...],
                     preferred_element_type=jnp.float32)
    colf = lax.broadcasted_iota(
        jnp.int32, logits.shape, 1).astype(jnp.float32)
    if nt * vt != vocab:
        limit = (vocab - i * vt).astype(jnp.float32)
        masked = jnp.where(colf < limit, logits, _NEG)
    else:
        masked = logits

    tile_m = jnp.max(masked, axis=1, keepdims=True)
    new_m = jnp.maximum(m_s[...], tile_m)
    e = jnp.exp(masked - new_m)
    s_s[...] = (s_s[...] * jnp.exp(m_s[...] - new_m)
                + jnp.sum(e, axis=1, keepdims=True))
    m_s[...] = new_m

    tv = tv_s[...]
    ti = ti_s[...]
    v1, v2, v3 = tv[:, 0:1], tv[:, 1:2], tv[:, 2:3]
    i1, i2, i3 = ti[:, 0:1], ti[:, 1:2], ti[:, 2:3]
    cur = masked
    cm = tile_m
    for r in range(KB):
        eq = cur == cm
        cidxf = jnp.min(jnp.where(eq, colf, float(vt)),
                        axis=1, keepdims=True)
        cidx = i * vt + cidxf.astype(jnp.int32)
        g1, g2, g3 = cm > v1, cm > v2, cm > v3
        v1, v2, v3 = (jnp.where(g1, cm, v1),
                      jnp.where(g1, v1, jnp.where(g2, cm, v2)),
                      jnp.where(g2, v2, jnp.where(g3, cm, v3)))
        i1, i2, i3 = (jnp.where(g1, cidx, i1),
                      jnp.where(g1, i1, jnp.where(g2, cidx, i2)),
                      jnp.where(g2, i2, jnp.where(g3, cidx, i3)))
        if r < KB - 1:
            cur = jnp.where(colf == cidxf, _NEG, cur)
            cm = jnp.max(cur, axis=1, keepdims=True)
    tv_s[...] = jnp.concatenate([v1, v2, v3], axis=1)
    ti_s[...] = jnp.concatenate([i1, i2, i3], axis=1)

    @pl.when(i == nt - 1)
    def _():
        tv_ref[...] = tv_s[...]
        ti_ref[...] = ti_s[...]
        lse_ref[...] = m_s[...] + jnp.log(s_s[...])


def _fused_step(emb, ctx_rows, W_h, W_out, vt=2048):
    r, d = emb.shape
    vocab = W_out.shape[1]
    nt = pl.cdiv(vocab, vt)
    full = lambda i: (0, 0)
    return pl.pallas_call(
        functools.partial(_step_body, nt=nt, vt=vt, vocab=vocab),
        grid=(nt,),
        in_specs=[
            pl.BlockSpec((r, d), full),
            pl.BlockSpec((r, d), full),
            pl.BlockSpec((d, d), full),
            pl.BlockSpec((d, vt), lambda i: (0, i)),
        ],
        out_specs=[
            pl.BlockSpec((r, KB), full),
            pl.BlockSpec((r, KB), full),
            pl.BlockSpec((r, 1), full),
        ],
        out_shape=[
            jax.ShapeDtypeStruct((r, KB), jnp.float32),
            jax.ShapeDtypeStruct((r, KB), jnp.int32),
            jax.ShapeDtypeStruct((r, 1), jnp.float32),
        ],
        scratch_shapes=[
            pltpu.VMEM((r, d), jnp.float32),
            pltpu.VMEM((r, 1), jnp.float32),
            pltpu.VMEM((r, 1), jnp.float32),
            pltpu.VMEM((r, KB), jnp.float32),
            pltpu.VMEM((r, KB), jnp.int32),
        ],
    )(emb, ctx_rows, W_h, W_out)


def _merge_body(tv_ref, ti_ref, lse_ref, cum_ref, beams_ref,
                ncum_ref, nbeams_ref, tok_ref, *, t):
    cand = ((tv_ref[...] - lse_ref[...][:, :, None])
            + cum_ref[...][:, :, None])
    fio = (lax.broadcasted_iota(jnp.int32, cand.shape, 1) * KB
           + lax.broadcasted_iota(jnp.int32, cand.shape, 2))
    ti = ti_ref[...]
    beams = beams_ref[...]
    pos = lax.broadcasted_iota(jnp.int32, beams.shape[::2], 1)

    cur = cand
    vals, toks, rows = [], [], []
    for _ in range(KB):
        m2 = jnp.max(cur, axis=2, keepdims=True)
        m = jnp.max(m2, axis=1, keepdims=True)
        eq = cur == m
        fi = jnp.min(jnp.where(eq, fio, _IMAX), axis=(1, 2))
        sel = fio == fi[:, None, None]
        vals.append(m[:, :, 0])
        tok = jnp.sum(jnp.where(sel, ti, 0), axis=(1, 2))
        toks.append(tok)
        prev = fi // KB
        row = jnp.zeros(beams.shape[::2], beams.dtype)
        for j in range(KB):
            row = jnp.where((prev == j)[:, None], beams[:, j, :], row)
        row = jnp.where(pos == t, tok[:, None], row)
        rows.append(row[:, None, :])
        cur = jnp.where(sel, _NEG, cur)

    ncum_ref[...] = jnp.concatenate(vals, axis=1)
    tok_ref[...] = jnp.stack(toks, axis=1)
    nbeams_ref[...] = jnp.concatenate(rows, axis=1)


def _merge_step(tv, ti, lse, cum_ps, beams, t):
    b, _, tmax = beams.shape
    return pl.pallas_call(
        functools.partial(_merge_body, t=t),
        out_shape=[
            jax.ShapeDtypeStruct((b, KB), jnp.float32),
            jax.ShapeDtypeStruct((b, KB, tmax), jnp.int32),
            jax.ShapeDtypeStruct((b, KB), jnp.int32),
        ],
    )(tv.reshape(b, KB, KB), ti.reshape(b, KB, KB),
      lse.reshape(b, KB), cum_ps, beams)


def kernel(input_var, encoder_outputs, k, W_emb, W_h, W_out):
    bsz = encoder_outputs.shape[0]
    tdec = 8

    ctx = _pooled_ctx(encoder_outputs)
    vt = 2048

    emb0 = jnp.take(W_emb, input_var[:, 0], axis=0)
    tv, ti, lse = _fused_step(emb0, ctx, W_h, W_out, vt=vt)
    cum_ps = tv - lse
    beams = jnp.zeros((bsz, KB, tdec), jnp.int32)
    beams = beams.at[:, :, 0].set(ti)
    last = ti.reshape(bsz * KB)

    ctx_k = jnp.repeat(ctx, KB, axis=0)
    for t in range(1, tdec):
        emb = jnp.take(W_emb, last, axis=0)
        tv, ti, lse = _fused_step(emb, ctx_k, W_h, W_out, vt=vt)
        cum_ps, beams, tok = _merge_step(tv, ti, lse, cum_ps, beams, t)
        last = tok.reshape(bsz * KB)

    return beams[:, 0, :], cum_ps


# speedup vs baseline: 1.2319x; 1.2319x over previous
import functools

import jax
import jax.numpy as jnp
from jax import lax
from jax.experimental import pallas as pl
from jax.experimental.pallas import tpu as pltpu
from jax.experimental.pallas import tpu_sc as plsc

KB = 3
_NEG = -jnp.inf
_IMAX = 2**31 - 1


def _ctx_body(enc_ref, out_ref, acc_s, *, nc, inv):
    j = pl.program_id(1)

    @pl.when(j == 0)
    def _():
        acc_s[...] = jnp.zeros_like(acc_s)

    acc_s[...] += jnp.sum(enc_ref[...], axis=1)

    @pl.when(j == nc - 1)
    def _():
        out_ref[...] = acc_s[...] * inv


def _pooled_ctx(enc):
    b, seq, d = enc.shape
    bb, ch = 8, 256
    nc = seq // ch
    return pl.pallas_call(
        functools.partial(_ctx_body, nc=nc, inv=1.0 / seq),
        grid=(b // bb, nc),
        in_specs=[pl.BlockSpec((bb, ch, d), lambda i, j: (i, j, 0))],
        out_specs=pl.BlockSpec((bb, d), lambda i, j: (i, 0)),
        out_shape=jax.ShapeDtypeStruct((b, d), jnp.float32),
        scratch_shapes=[pltpu.VMEM((bb, d), jnp.float32)],
    )(enc)


def _group3(x, rmod):
    xm1 = jnp.roll(x, 1, axis=0)
    xm2 = jnp.roll(x, 2, axis=0)
    xp1 = jnp.roll(x, -1, axis=0)
    xp2 = jnp.roll(x, -2, axis=0)
    a0 = jnp.where(rmod == 0, x, jnp.where(rmod == 1, xm1, xm2))
    a1 = jnp.where(rmod == 0, xp1, jnp.where(rmod == 1, x, xm1))
    a2 = jnp.where(rmod == 0, xp2, jnp.where(rmod == 1, xp1, x))
    return a0, a1, a2


def _step_body(emb_ref, ctx_ref, wh_ref, wout_ref, wemb_ref, cum_ref,
               beams_ref, ncum_ref, nbeams_ref, tok_ref, nemb_ref,
               h_s, m_s, s_s, tv_s, ti_s, er_s, sem,
               *, nt, vt, vocab, t, last_step):
    i = pl.program_id(0)

    @pl.when(i == 0)
    def _():
        x = emb_ref[...] + ctx_ref[...]
        h_s[...] = jnp.tanh(jnp.dot(x, wh_ref[...],
                                    preferred_element_type=jnp.float32))
        m_s[...] = jnp.full_like(m_s, _NEG)
        s_s[...] = jnp.zeros_like(s_s)
        tv_s[...] = jnp.full_like(tv_s, _NEG)
        ti_s[...] = jnp.zeros_like(ti_s)

    logits = jnp.dot(h_s[...], wout_ref[...],
                     preferred_element_type=jnp.float32)
    colf = lax.broadcasted_iota(
        jnp.int32, logits.shape, 1).astype(jnp.float32)
    if nt * vt != vocab:
        limit = (vocab - i * vt).astype(jnp.float32)
        masked = jnp.where(colf < limit, logits, _NEG)
    else:
        masked = logits

    tile_m = jnp.max(masked, axis=1, keepdims=True)
    new_m = jnp.maximum(m_s[...], tile_m)
    e = jnp.exp(masked - new_m)
    s_s[...] = (s_s[...] * jnp.exp(m_s[...] - new_m)
                + jnp.sum(e, axis=1, keepdims=True))
    m_s[...] = new_m

    tv = tv_s[...]
    ti = ti_s[...]
    v1, v2, v3 = tv[:, 0:1], tv[:, 1:2], tv[:, 2:3]
    i1, i2, i3 = ti[:, 0:1], ti[:, 1:2], ti[:, 2:3]
    cur = masked
    cm = tile_m
    for r in range(KB):
        eq = cur == cm
        cidxf = jnp.min(jnp.where(eq, colf, float(vt)),
                        axis=1, keepdims=True)
        cidx = i * vt + cidxf.astype(jnp.int32)
        g1, g2, g3 = cm > v1, cm > v2, cm > v3
        v1, v2, v3 = (jnp.where(g1, cm, v1),
                      jnp.where(g1, v1, jnp.where(g2, cm, v2)),
                      jnp.where(g2, v2, jnp.where(g3, cm, v3)))
        i1, i2, i3 = (jnp.where(g1, cidx, i1),
                      jnp.where(g1, i1, jnp.where(g2, cidx, i2)),
                      jnp.where(g2, i2, jnp.where(g3, cidx, i3)))
        if r < KB - 1:
            cur = jnp.where(colf == cidxf, _NEG, cur)
            cm = jnp.max(cur, axis=1, keepdims=True)
    tv_s[...] = jnp.concatenate([v1, v2, v3], axis=1)
    ti_s[...] = jnp.concatenate([i1, i2, i3], axis=1)

    @pl.when(i == nt - 1)
    def _():
        lse = m_s[...] + jnp.log(s_s[...])
        cum = cum_ref[...]
        cand = (tv_s[...] - lse) + cum
        rows = lax.broadcasted_iota(jnp.int32, cand.shape, 0)
        rmod = rows - (rows // KB) * KB
        rmod1 = rmod[:, 0:1]
        c0, c1, c2 = _group3(cand, rmod)
        all9 = jnp.concatenate([c0, c1, c2], axis=1)
        ti = ti_s[...]
        t0, t1, t2 = _group3(ti, rmod)
        ti9 = jnp.concatenate([t0, t1, t2], axis=1)
        lane9 = lax.broadcasted_iota(jnp.int32, all9.shape, 1)

        beams = beams_ref[...]
        rmodb = rmod1 + jnp.zeros_like(beams)
        b0, b1, b2 = _group3(beams, rmodb)
        pos = lax.broadcasted_iota(jnp.int32, beams.shape, 1)

        cur = all9
        vs, ts, ps = [], [], []
        for _ in range(KB):
            m = jnp.max(cur, axis=1, keepdims=True)
            eq = cur == m
            fi = jnp.min(jnp.where(eq, lane9, _IMAX),
                         axis=1, keepdims=True)
            sel = lane9 == fi
            vs.append(m)
            ts.append(jnp.sum(jnp.where(sel, ti9, 0),
                              axis=1, keepdims=True))
            ps.append(fi // KB)
            cur = jnp.where(sel, _NEG, cur)

        pick = lambda xs: jnp.where(
            rmod1 == 0, xs[0], jnp.where(rmod1 == 1, xs[1], xs[2]))
        ncum_ref[...] = pick(vs)
        tok = pick(ts)
        tok_ref[...] = tok
        prev = pick(ps)
        nb = jnp.where(prev == 0, b0, jnp.where(prev == 1, b1, b2))
        nbeams_ref[...] = jnp.where(pos == t, tok, nb)

        if not last_step:
            nrows = tok.shape[0]
            tokf = tok.astype(jnp.float32)
            riota = lax.broadcasted_iota(jnp.int32, tokf.shape, 0)
            copies = []
            for rr in range(nrows):
                s = jnp.sum(jnp.where(riota == rr, tokf, 0.0))
                si = s.astype(jnp.int32)
                c = pltpu.make_async_copy(
                    wemb_ref.at[pl.ds(si, 1), :],
                    er_s.at[pl.ds(rr, 1), :], sem)
                c.start()
                copies.append(c)
            for c in copies:
                c.wait()
            nemb_ref[...] = er_s[...]


def _fused_step(emb, ctx_rows, W_h, W_out, W_emb, cum_row, beams, t,
                vt=2048):
    r, d = emb.shape
    vocab = W_out.shape[1]
    tdec = beams.shape[1]
    nt = pl.cdiv(vocab, vt)
    last_step = t == tdec - 1
    full = lambda i: (0, 0)
    return pl.pallas_call(
        functools.partial(_step_body, nt=nt, vt=vt, vocab=vocab, t=t,
                          last_step=last_step),
        grid=(nt,),
        in_specs=[
            pl.BlockSpec((r, d), full),
            pl.BlockSpec((r, d), full),
            pl.BlockSpec((d, d), full),
            pl.BlockSpec((d, vt), lambda i: (0, i)),
            pl.BlockSpec(memory_space=pl.ANY),
            pl.BlockSpec((r, 1), full),
            pl.BlockSpec((r, tdec), full),
        ],
        out_specs=[
            pl.BlockSpec((r, 1), full),
            pl.BlockSpec((r, tdec), full),
            pl.BlockSpec((r, 1), full),
            pl.BlockSpec((r, d), full),
        ],
        out_shape=[
            jax.ShapeDtypeStruct((r, 1), jnp.float32),
            jax.ShapeDtypeStruct((r, tdec), jnp.int32),
            jax.ShapeDtypeStruct((r, 1), jnp.int32),
            jax.ShapeDtypeStruct((r, d), jnp.float32),
        ],
        scratch_shapes=[
            pltpu.VMEM((r, d), jnp.float32),
            pltpu.VMEM((r, 1), jnp.float32),
            pltpu.VMEM((r, 1), jnp.float32),
            pltpu.VMEM((r, KB), jnp.float32),
            pltpu.VMEM((r, KB), jnp.int32),
            pltpu.VMEM((r, d), jnp.float32),
            pltpu.SemaphoreType.DMA,
        ],
    )(emb, ctx_rows, W_h, W_out, W_emb, cum_row, beams)


def _sc_gather(tok, table):
    r = tok.shape[0]
    d = table.shape[1]
    nw = r // 8
    mesh = plsc.VectorSubcoreMesh(core_axis_name="c", subcore_axis_name="s")

    @functools.partial(
        pl.kernel,
        out_type=jax.ShapeDtypeStruct((r, d), jnp.float32),
        mesh=mesh,
        scratch_types=[pltpu.VMEM((8,), jnp.int32),
                       pltpu.VMEM((8, d), jnp.float32),
                       pltpu.SemaphoreType.DMA],
    )
    def gk(tok_hbm, table_hbm, out_hbm, idx_v, rows_v, sem):
        wid = lax.axis_index("s") * 2 + lax.axis_index("c")

        @pl.when(wid < nw)
        def _():
            base = wid * 8
            pltpu.sync_copy(tok_hbm.at[pl.ds(base, 8)], idx_v)
            pltpu.async_copy(table_hbm.at[idx_v], rows_v, sem).wait()
            pltpu.sync_copy(rows_v, out_hbm.at[pl.ds(base, 8)])

    return gk(tok, table)


def kernel(input_var, encoder_outputs, k, W_emb, W_h, W_out):
    bsz = encoder_outputs.shape[0]
    tdec = 8
    r = bsz * KB

    ctx = _pooled_ctx(encoder_outputs)
    ctx_k = jnp.repeat(ctx, KB, axis=0)
    vt = 2048

    cum_row = jnp.where(jnp.arange(r) % KB == 0,
                        0.0, -jnp.inf)[:, None].astype(jnp.float32)
    beams = jnp.zeros((r, tdec), jnp.int32)
    last = jnp.repeat(input_var[:, 0], KB)

    emb = _sc_gather(last, W_emb)
    for t in range(tdec):
        cum_row, beams, tok, emb = _fused_step(
            emb, ctx_k, W_h, W_out, W_emb, cum_row, beams, t, vt=vt)

    hyp = beams.reshape(bsz, KB, tdec)[:, 0, :]
    return hyp, cum_row.reshape(bsz, KB)


# speedup vs baseline: 1.5478x; 1.2565x over previous
import functools

import jax
import jax.numpy as jnp
from jax import lax
from jax.experimental import pallas as pl
from jax.experimental.pallas import tpu as pltpu
from jax.experimental.pallas import tpu_sc as plsc

KB = 3
_NEG = -jnp.inf
_IMAX = 2**31 - 1


def _ctx_body(enc_ref, out_ref, acc_s, *, nc, inv):
    j = pl.program_id(1)

    @pl.when(j == 0)
    def _():
        acc_s[...] = jnp.zeros_like(acc_s)

    acc_s[...] += jnp.sum(enc_ref[...], axis=1)

    @pl.when(j == nc - 1)
    def _():
        out_ref[...] = acc_s[...] * inv


def _pooled_ctx(enc):
    b, seq, d = enc.shape
    bb, ch = 8, 256
    nc = seq // ch
    return pl.pallas_call(
        functools.partial(_ctx_body, nc=nc, inv=1.0 / seq),
        grid=(b // bb, nc),
        in_specs=[pl.BlockSpec((bb, ch, d), lambda i, j: (i, j, 0))],
        out_specs=pl.BlockSpec((bb, d), lambda i, j: (i, 0)),
        out_shape=jax.ShapeDtypeStruct((b, d), jnp.float32),
        scratch_shapes=[pltpu.VMEM((bb, d), jnp.float32)],
    )(enc)


def _group3(x, rmod):
    xm1 = jnp.roll(x, 1, axis=0)
    xm2 = jnp.roll(x, 2, axis=0)
    xp1 = jnp.roll(x, -1, axis=0)
    xp2 = jnp.roll(x, -2, axis=0)
    a0 = jnp.where(rmod == 0, x, jnp.where(rmod == 1, xm1, xm2))
    a1 = jnp.where(rmod == 0, xp1, jnp.where(rmod == 1, x, xm1))
    a2 = jnp.where(rmod == 0, xp2, jnp.where(rmod == 1, xp1, x))
    return a0, a1, a2


def _step_body(emb_ref, ctx_ref, wh_ref, wout_ref, cum_ref, beams_ref,
               ncum_ref, nbeams_ref, tok_ref,
               h_s, m_s, s_s, tv_s, ti_s, *, nt, vt, vocab, t):
    i = pl.program_id(0)

    @pl.when(i == 0)
    def _():
        x = emb_ref[...] + ctx_ref[...]
        h_s[...] = jnp.tanh(jnp.dot(x, wh_ref[...],
                                    preferred_element_type=jnp.float32))
        m_s[...] = jnp.full_like(m_s, _NEG)
        s_s[...] = jnp.zeros_like(s_s)
        tv_s[...] = jnp.full_like(tv_s, _NEG)
        ti_s[...] = jnp.zeros_like(ti_s)

    logits = jnp.dot(h_s[...], wout_ref[...],
                     preferred_element_type=jnp.float32)
    colf = lax.broadcasted_iota(
        jnp.int32, logits.shape, 1).astype(jnp.float32)
    if nt * vt != vocab:
        limit = (vocab - i * vt).astype(jnp.float32)
        masked = jnp.where(colf < limit, logits, _NEG)
    else:
        masked = logits

    tile_m = jnp.max(masked, axis=1, keepdims=True)
    new_m = jnp.maximum(m_s[...], tile_m)
    e = jnp.exp(masked - new_m)
    s_s[...] = (s_s[...] * jnp.exp(m_s[...] - new_m)
                + jnp.sum(e, axis=1, keepdims=True))
    m_s[...] = new_m

    tv = tv_s[...]
    ti = ti_s[...]
    v1, v2, v3 = tv[:, 0:1], tv[:, 1:2], tv[:, 2:3]
    i1, i2, i3 = ti[:, 0:1], ti[:, 1:2], ti[:, 2:3]
    cur = masked
    cm = tile_m
    for r in range(KB):
        eq = cur == cm
        cidxf = jnp.min(jnp.where(eq, colf, float(vt)),
                        axis=1, keepdims=True)
        cidx = i * vt + cidxf.astype(jnp.int32)
        g1, g2, g3 = cm > v1, cm > v2, cm > v3
        v1, v2, v3 = (jnp.where(g1, cm, v1),
                      jnp.where(g1, v1, jnp.where(g2, cm, v2)),
                      jnp.where(g2, v2, jnp.where(g3, cm, v3)))
        i1, i2, i3 = (jnp.where(g1, cidx, i1),
                      jnp.where(g1, i1, jnp.where(g2, cidx, i2)),
                      jnp.where(g2, i2, jnp.where(g3, cidx, i3)))
        if r < KB - 1:
            cur = jnp.where(colf == cidxf, _NEG, cur)
            cm = jnp.max(cur, axis=1, keepdims=True)
    tv_s[...] = jnp.concatenate([v1, v2, v3], axis=1)
    ti_s[...] = jnp.concatenate([i1, i2, i3], axis=1)

    @pl.when(i == nt - 1)
    def _():
        lse = m_s[...] + jnp.log(s_s[...])
        cum = cum_ref[...]
        cand = (tv_s[...] - lse) + cum
        rows = lax.broadcasted_iota(jnp.int32, cand.shape, 0)
        rmod = rows - (rows // KB) * KB
        rmod1 = rmod[:, 0:1]
        c0, c1, c2 = _group3(cand, rmod)
        all9 = jnp.concatenate([c0, c1, c2], axis=1)
        ti = ti_s[...]
        t0, t1, t2 = _group3(ti, rmod)
        ti9 = jnp.concatenate([t0, t1, t2], axis=1)
        lane9 = lax.broadcasted_iota(jnp.int32, all9.shape, 1)

        beams = beams_ref[...]
        rmodb = rmod1 + jnp.zeros_like(beams)
        b0, b1, b2 = _group3(beams, rmodb)
        pos = lax.broadcasted_iota(jnp.int32, beams.shape, 1)

        cur = all9
        vs, ts, ps = [], [], []
        for _ in range(KB):
            m = jnp.max(cur, axis=1, keepdims=True)
            eq = cur == m
            fi = jnp.min(jnp.where(eq, lane9, _IMAX),
                         axis=1, keepdims=True)
            sel = lane9 == fi
            vs.append(m)
            ts.append(jnp.sum(jnp.where(sel, ti9, 0),
                              axis=1, keepdims=True))
            ps.append(fi // KB)
            cur = jnp.where(sel, _NEG, cur)

        pick = lambda xs: jnp.where(
            rmod1 == 0, xs[0], jnp.where(rmod1 == 1, xs[1], xs[2]))
        ncum_ref[...] = pick(vs)
        tok = pick(ts)
        tok_ref[...] = tok
        prev = pick(ps)
        nb = jnp.where(prev == 0, b0, jnp.where(prev == 1, b1, b2))
        nbeams_ref[...] = jnp.where(pos == t, tok, nb)


def _fused_step(emb, ctx_rows, W_h, W_out, cum_row, beams, t, vt=2048):
    r, d = emb.shape
    vocab = W_out.shape[1]
    tdec = beams.shape[1]
    nt = pl.cdiv(vocab, vt)
    full = lambda i: (0, 0)
    return pl.pallas_call(
        functools.partial(_step_body, nt=nt, vt=vt, vocab=vocab, t=t),
        grid=(nt,),
        in_specs=[
            pl.BlockSpec((r, d), full),
            pl.BlockSpec((r, d), full),
            pl.BlockSpec((d, d), full),
            pl.BlockSpec((d, vt), lambda i: (0, i)),
            pl.BlockSpec((r, 1), full),
            pl.BlockSpec((r, tdec), full),
        ],
        out_specs=[
            pl.BlockSpec((r, 1), full),
            pl.BlockSpec((r, tdec), full),
            pl.BlockSpec((r, 1), full),
        ],
        out_shape=[
            jax.ShapeDtypeStruct((r, 1), jnp.float32),
            jax.ShapeDtypeStruct((r, tdec), jnp.int32),
            jax.ShapeDtypeStruct((r, 1), jnp.int32),
        ],
        scratch_shapes=[
            pltpu.VMEM((r, d), jnp.float32),
            pltpu.VMEM((r, 1), jnp.float32),
            pltpu.VMEM((r, 1), jnp.float32),
            pltpu.VMEM((r, KB), jnp.float32),
            pltpu.VMEM((r, KB), jnp.int32),
        ],
    )(emb, ctx_rows, W_h, W_out, cum_row, beams)


def _sc_gather(tok, table):
    r = tok.shape[0]
    d = table.shape[1]
    nw = r // 8
    mesh = plsc.VectorSubcoreMesh(core_axis_name="c", subcore_axis_name="s")

    @functools.partial(
        pl.kernel,
        out_type=jax.ShapeDtypeStruct((r, d), jnp.float32),
        mesh=mesh,
        scratch_types=[pltpu.VMEM((8,), jnp.int32),
                       pltpu.VMEM((8, d), jnp.float32),
                       pltpu.SemaphoreType.DMA],
    )
    def gk(tok_hbm, table_hbm, out_hbm, idx_v, rows_v, sem):
        wid = lax.axis_index("s") * 2 + lax.axis_index("c")

        @pl.when(wid < nw)
        def _():
            base = wid * 8
            pltpu.sync_copy(tok_hbm.at[pl.ds(base, 8)], idx_v)
            pltpu.async_copy(table_hbm.at[idx_v], rows_v, sem).wait()
            pltpu.sync_copy(rows_v, out_hbm.at[pl.ds(base, 8)])

    return gk(tok, table)


def kernel(input_var, encoder_outputs, k, W_emb, W_h, W_out):
    bsz = encoder_outputs.shape[0]
    tdec = 8
    r = bsz * KB

    ctx = _pooled_ctx(encoder_outputs)
    ctx_k = jnp.repeat(ctx, KB, axis=0)
    vt = 4096

    cum_row = jnp.where(jnp.arange(r) % KB == 0,
                        0.0, -jnp.inf)[:, None].astype(jnp.float32)
    beams = jnp.zeros((r, tdec), jnp.int32)
    last = jnp.repeat(input_var[:, 0], KB)

    for t in range(tdec):
        emb = _sc_gather(last, W_emb)
        cum_row, beams, tok = _fused_step(
            emb, ctx_k, W_h, W_out, cum_row, beams, t, vt=vt)
        last = tok.reshape(r)

    hyp = beams.reshape(bsz, KB, tdec)[:, 0, :]
    return hyp, cum_row.reshape(bsz, KB)
